# baseline (speedup 1.0000x reference)
import jax
import jax.numpy as jnp
from jax.experimental import pallas as pl
from jax.experimental.pallas import tpu as pltpu

WINDOW = 512
HEAD_DIM = 64
NUM_HEADS = 12
BQ = 256
KB = WINDOW + BQ


def _attn_kernel(x_ref, o_ref, q_s, k_s, v_s, q_sem, k_sem, v_sem):
    ib = pl.program_id(0)
    i = pl.program_id(1)
    scale = 1.0 / (HEAD_DIM ** 0.5)
    kstart = jnp.maximum(i - 2, 0) * BQ

    q_cp = pltpu.make_async_copy(
        x_ref.at[ib, pl.ds(i * BQ, BQ), 0], q_s, q_sem)
    q_cp.start()

    @pl.when(i == 0)
    def _load_panels():
        k_cp = pltpu.make_async_copy(x_ref.at[ib, :, 1], k_s, k_sem)
        v_cp = pltpu.make_async_copy(x_ref.at[ib, :, 2], v_s, v_sem)
        k_cp.start()
        v_cp.start()
        k_cp.wait()
        v_cp.wait()

    q_cp.wait()

    q_idx = i * BQ + jax.lax.broadcasted_iota(jnp.int32, (BQ, KB), 0)
    kv_idx = kstart + jax.lax.broadcasted_iota(jnp.int32, (BQ, KB), 1)
    diff = q_idx - kv_idx
    mask = (diff >= 0) & (diff <= WINDOW)
    bias = jnp.where(mask, jnp.float32(0), jnp.float32(float("-inf")))
    for hh in range(NUM_HEADS):
        qh = q_s[:, hh, :] * scale
        kh = k_s[pl.ds(kstart, KB), hh, :]
        vh = v_s[pl.ds(kstart, KB), hh, :]
        s = jax.lax.dot_general(
            qh, kh, (((1,), (1,)), ((), ())),
            preferred_element_type=jnp.float32) + bias
        m = jnp.max(s, axis=-1, keepdims=True)
        p = jnp.exp(s - m)
        denom = jnp.sum(p, axis=-1, keepdims=True)
        oh = jax.lax.dot_general(
            p, vh, (((1,), (0,)), ((), ())),
            preferred_element_type=jnp.float32)
        o_ref[0, :, hh, :] = oh * (1.0 / denom)


def kernel(qkv):
    b, l, three, h, e = qkv.shape
    nq = l // BQ

    return pl.pallas_call(
        _attn_kernel,
        grid=(b, nq),
        in_specs=[pl.BlockSpec(memory_space=pltpu.MemorySpace.HBM)],
        out_specs=pl.BlockSpec((1, BQ, h, e), lambda ib, i: (ib, i, 0, 0)),
        out_shape=jax.ShapeDtypeStruct((b, l, h, e), jnp.float32),
        scratch_shapes=[
            pltpu.VMEM((BQ, h, e), jnp.float32),
            pltpu.VMEM((l, h, e), jnp.float32),
            pltpu.VMEM((l, h, e), jnp.float32),
            pltpu.SemaphoreType.DMA,
            pltpu.SemaphoreType.DMA,
            pltpu.SemaphoreType.DMA,
        ],
    )(qkv)


# speedup vs baseline: 2.0984x; 2.0984x over previous
import jax
import jax.numpy as jnp
from jax.experimental import pallas as pl

WINDOW = 512
HEAD_DIM = 64
NUM_HEADS = 12
BQ = 256
KB = WINDOW + BQ


def _attn_kernel(q_ref, k_ref, v_ref, o_ref):
    i = pl.program_id(1)
    scale = 1.0 / (HEAD_DIM ** 0.5)
    kstart = jnp.maximum(i - 2, 0) * BQ
    q_idx = i * BQ + jax.lax.broadcasted_iota(jnp.int32, (BQ, KB), 0)
    kv_idx = kstart + jax.lax.broadcasted_iota(jnp.int32, (BQ, KB), 1)
    diff = q_idx - kv_idx
    mask = (diff >= 0) & (diff <= WINDOW)
    bias = jnp.where(mask, jnp.float32(0), jnp.float32(float("-inf")))
    for hh in range(NUM_HEADS):
        c0 = hh * HEAD_DIM
        qh = q_ref[0, :, c0:c0 + HEAD_DIM]
        kh = k_ref[0, pl.ds(kstart, KB), c0:c0 + HEAD_DIM]
        vh = v_ref[0, pl.ds(kstart, KB), c0:c0 + HEAD_DIM]
        s = jax.lax.dot_general(
            qh, kh, (((1,), (1,)), ((), ())),
            preferred_element_type=jnp.float32) * scale + bias
        m = jnp.max(s, axis=-1, keepdims=True)
        p = jnp.exp(s - m)
        denom = jnp.sum(p, axis=-1, keepdims=True)
        oh = jax.lax.dot_general(
            p.astype(jnp.bfloat16), vh, (((1,), (0,)), ((), ())),
            preferred_element_type=jnp.float32)
        o_ref[0, :, c0:c0 + HEAD_DIM] = oh * (1.0 / denom)


def kernel(qkv):
    b, l, three, h, e = qkv.shape
    ch = h * e
    x = qkv.reshape(b, l, three * ch).astype(jnp.bfloat16)
    nq = l // BQ

    out = pl.pallas_call(
        _attn_kernel,
        grid=(b, nq),
        in_specs=[
            pl.BlockSpec((1, BQ, ch), lambda ib, i: (ib, i, 0)),
            pl.BlockSpec((1, l, ch), lambda ib, i: (ib, 0, 1)),
            pl.BlockSpec((1, l, ch), lambda ib, i: (ib, 0, 2)),
        ],
        out_specs=pl.BlockSpec((1, BQ, ch), lambda ib, i: (ib, i, 0)),
        out_shape=jax.ShapeDtypeStruct((b, l, ch), jnp.float32),
    )(x, x, x)

    return out.reshape(b, l, h, e)


# speedup vs baseline: 2.3022x; 1.0971x over previous
import jax
import jax.numpy as jnp
from jax.experimental import pallas as pl
from jax.experimental.pallas import tpu as pltpu

WINDOW = 512
HEAD_DIM = 64
NUM_HEADS = 12
BQ = 256
KB = WINDOW + BQ


def _attn_kernel(q_ref, k_ref, v_ref, o_ref):
    i = pl.program_id(1)
    scale = 1.0 / (HEAD_DIM ** 0.5)
    kstart = jnp.maximum(i - 2, 0) * BQ
    q_idx = i * BQ + jax.lax.broadcasted_iota(jnp.int32, (BQ, KB), 0)
    kv_idx = kstart + jax.lax.broadcasted_iota(jnp.int32, (BQ, KB), 1)
    diff = q_idx - kv_idx
    mask = (diff >= 0) & (diff <= WINDOW)
    bias = jnp.where(mask, jnp.float32(0), jnp.float32(float("-inf")))
    for hh in range(NUM_HEADS):
        c0 = hh * HEAD_DIM
        qh = q_ref[0, :, c0:c0 + HEAD_DIM]
        kh = k_ref[0, pl.ds(kstart, KB), c0:c0 + HEAD_DIM]
        vh = v_ref[0, pl.ds(kstart, KB), c0:c0 + HEAD_DIM]
        s = jax.lax.dot_general(
            qh, kh, (((1,), (1,)), ((), ())),
            preferred_element_type=jnp.float32) * scale + bias
        m = jnp.max(s, axis=-1, keepdims=True)
        p = jnp.exp(s - m)
        denom = jnp.sum(p, axis=-1, keepdims=True)
        oh = jax.lax.dot_general(
            p, vh, (((1,), (0,)), ((), ())),
            preferred_element_type=jnp.float32)
        o_ref[0, :, c0:c0 + HEAD_DIM] = oh * (1.0 / denom)


def kernel(qkv):
    b, l, three, h, e = qkv.shape
    ch = h * e
    x = qkv.reshape(b, l, three * ch)
    nq = l // BQ

    out = pl.pallas_call(
        _attn_kernel,
        grid=(b, nq),
        in_specs=[
            pl.BlockSpec((1, BQ, ch), lambda ib, i: (ib, i, 0)),
            pl.BlockSpec((1, l, ch), lambda ib, i: (ib, 0, 1)),
            pl.BlockSpec((1, l, ch), lambda ib, i: (ib, 0, 2)),
        ],
        out_specs=pl.BlockSpec((1, BQ, ch), lambda ib, i: (ib, i, 0)),
        out_shape=jax.ShapeDtypeStruct((b, l, ch), jnp.float32),
        compiler_params=pltpu.CompilerParams(
            dimension_semantics=("parallel", "arbitrary"),
            allow_input_fusion=[True, True, True],
        ),
    )(x, x, x)

    return out.reshape(b, l, h, e)


# speedup vs baseline: 2.3045x; 1.0010x over previous
import jax
import jax.numpy as jnp
from jax.experimental import pallas as pl
from jax.experimental.pallas import tpu as pltpu

WINDOW = 512
HEAD_DIM = 64
NUM_HEADS = 12
BQ = 256
KB = WINDOW + BQ


def _attn_kernel(q_ref, k_ref, v_ref, o_ref):
    i = pl.program_id(1)
    scale = 1.0 / (HEAD_DIM ** 0.5)
    kstart = jnp.maximum(i - 2, 0) * BQ
    q_idx = i * BQ + jax.lax.broadcasted_iota(jnp.int32, (BQ, KB), 0)
    kv_idx = kstart + jax.lax.broadcasted_iota(jnp.int32, (BQ, KB), 1)
    diff = q_idx - kv_idx
    mask = (diff >= 0) & (diff <= WINDOW)
    bias = jnp.where(mask, jnp.float32(0), jnp.float32(float("-inf")))
    for hh in range(NUM_HEADS):
        c0 = hh * HEAD_DIM
        qh = q_ref[0, :, c0:c0 + HEAD_DIM]
        kh = k_ref[0, pl.ds(kstart, KB), c0:c0 + HEAD_DIM]
        vh = v_ref[0, pl.ds(kstart, KB), c0:c0 + HEAD_DIM]
        s = jax.lax.dot_general(
            qh, kh, (((1,), (1,)), ((), ())),
            preferred_element_type=jnp.float32) * scale + bias
        m = jnp.max(s, axis=-1, keepdims=True)
        p = jnp.exp(s - m)
        denom = jnp.sum(p, axis=-1, keepdims=True)
        oh = jax.lax.dot_general(
            p, vh, (((1,), (0,)), ((), ())),
            preferred_element_type=jnp.float32)
        o_ref[0, :, c0:c0 + HEAD_DIM] = oh * (1.0 / denom)


def kernel(qkv):
    b, l, three, h, e = qkv.shape
    ch = h * e
    x = qkv.reshape(b, l, three * ch)
    nq = l // BQ

    out = pl.pallas_call(
        _attn_kernel,
        grid=(b, nq),
        in_specs=[
            pl.BlockSpec((1, BQ, ch), lambda ib, i: (ib, i, 0)),
            pl.BlockSpec((1, l, ch), lambda ib, i: (ib, 0, 1)),
            pl.BlockSpec((1, l, ch), lambda ib, i: (ib, 0, 2)),
        ],
        out_specs=pl.BlockSpec((1, BQ, ch), lambda ib, i: (ib, i, 0)),
        out_shape=jax.ShapeDtypeStruct((b, l, ch), jnp.float32),
        compiler_params=pltpu.CompilerParams(
            dimension_semantics=("parallel", "arbitrary"),
        ),
    )(x, x, x)

    return out.reshape(b, l, h, e)
